# baseline (speedup 1.0000x reference)
import jax
import jax.numpy as jnp
from jax import lax
from jax.experimental import pallas as pl
from jax.experimental.pallas import tpu as pltpu
from jax.experimental.pallas import tpu_sc as plsc

N = 10000
D = 128
E = 320000
HOPS = 8
ALPHA = 0.1

NC, NS, L = 2, 16, 16
F = 96
CHUNK = 128
EPT = 20096
NCHUNK = EPT // CHUNK
EPAD = EPT * NS
NPT = 640
RQ = 64
NQ = NPT // RQ
NPAD = NS * NPT
ZROW = N


def _rsqrt16(x):
    i = lax.bitcast_convert_type(x, jnp.int32)
    i = jnp.int32(0x5F3759DF) - lax.shift_right_logical(i, 1)
    y = lax.bitcast_convert_type(i, jnp.float32)
    for _ in range(3):
        y = y * (1.5 - 0.5 * x * y * y)
    return y


def _body(v0f, srcs, dsts, out, u_flat, srcp, dstp, agg, dinvrep,
          sbuf, dbuf, gbuf, dvbuf, abuf, ubuf, vbuf, zbuf, gsem):
    c = lax.axis_index("c")
    s = lax.axis_index("s")
    uoff = c * NPAD
    rb = s * NPT

    zero16 = jnp.zeros((L,), jnp.float32)
    one16 = jnp.full((L,), 1.0, jnp.float32)

    def zero_row96(i, _):
        for kk in range(F // L):
            zbuf[i, pl.ds(kk * L, L)] = zero16
        return 0
    lax.fori_loop(0, RQ, zero_row96, 0)

    def zero_row16(i, _):
        dvbuf[i, :] = zero16
        return 0
    lax.fori_loop(0, CHUNK, zero_row16, 0)

    for q in range(NPT // CHUNK):
        pltpu.sync_copy(dvbuf, dinvrep.at[pl.ds(rb + q * CHUNK, CHUNK)])
    for q in range(NQ):
        pltpu.sync_copy(zbuf, agg.at[pl.ds(rb + q * RQ, RQ)])
    plsc.subcore_barrier()

    def ones_row(i, _):
        dvbuf[i, :] = one16
        return 0
    lax.fori_loop(0, CHUNK, ones_row, 0)

    coff16 = jnp.broadcast_to(uoff, (L,)).astype(jnp.int32)

    def deg_chunk(j, _):
        pltpu.sync_copy(srcs.at[s, j], sbuf)
        pltpu.sync_copy(dsts.at[s, j], dbuf)
        for kk in range(CHUNK // L):
            sl = pl.ds(kk * L, L)
            s16 = sbuf[sl]
            d16 = dbuf[sl]
            valid = s16 != d16
            sbuf[sl] = jnp.where(valid, s16, ZROW) + coff16
            dbuf[sl] = jnp.where(valid, d16, ZROW)
        pltpu.sync_copy(dvbuf, dinvrep.at[dbuf], add=True)
        pltpu.sync_copy(sbuf, srcp.at[c, s, j])
        pltpu.sync_copy(dbuf, dstp.at[c, s, j])
        return 0
    lax.fori_loop(0, NCHUNK, deg_chunk, 0)
    plsc.subcore_barrier()

    def dinv_chunk(q, _):
        r = rb + q * RQ
        pltpu.sync_copy(dinvrep.at[pl.ds(r, RQ)], dvbuf.at[pl.ds(0, RQ)])

        def drow(i, _):
            dvbuf[i, :] = _rsqrt16(dvbuf[i, :] + 1.0)
            return 0
        lax.fori_loop(0, RQ, drow, 0)
        pltpu.sync_copy(dvbuf.at[pl.ds(0, RQ)], dinvrep.at[pl.ds(r, RQ)])
        return 0
    lax.fori_loop(0, NQ, dinv_chunk, 0)
    plsc.subcore_barrier()

    for q in range(NQ):
        r = rb + q * RQ
        pltpu.sync_copy(v0f.at[pl.ds(uoff + r, RQ)], vbuf)
        pltpu.sync_copy(dinvrep.at[pl.ds(r, RQ)], dvbuf.at[pl.ds(0, RQ)])

        def scale_row(i, _):
            dv = dvbuf[i, :]
            for kk in range(F // L):
                sl = pl.ds(kk * L, L)
                vbuf[i, sl] = vbuf[i, sl] * dv
            return 0
        lax.fori_loop(0, RQ, scale_row, 0)
        pltpu.sync_copy(vbuf, u_flat.at[pl.ds(uoff + r, RQ)])
    plsc.subcore_barrier()

    def scatter_phase():
        def chunk(j, _):
            pltpu.sync_copy(srcp.at[c, s, j], sbuf)
            pltpu.sync_copy(dstp.at[c, s, j], dbuf)
            pltpu.async_copy(u_flat.at[sbuf], gbuf, gsem).wait()
            pltpu.sync_copy(gbuf, agg.at[dbuf], add=True)
            return 0
        lax.fori_loop(0, NCHUNK, chunk, 0)

    def update_phase(final):
        a, b = jnp.float32(ALPHA), jnp.float32(1.0 - ALPHA)
        for q in range(NQ):
            r = rb + q * RQ
            pltpu.sync_copy(agg.at[pl.ds(r, RQ)], abuf)
            pltpu.sync_copy(zbuf, agg.at[pl.ds(r, RQ)])
            pltpu.sync_copy(u_flat.at[pl.ds(uoff + r, RQ)], ubuf)
            pltpu.sync_copy(v0f.at[pl.ds(uoff + r, RQ)], vbuf)
            pltpu.sync_copy(dinvrep.at[pl.ds(r, RQ)], dvbuf.at[pl.ds(0, RQ)])

            def row(i, _):
                dv = dvbuf[i, :]
                for kk in range(F // L):
                    sl = pl.ds(kk * L, L)
                    t = abuf[i, sl] + ubuf[i, sl]
                    if final:
                        res = a * vbuf[i, sl] + b * (dv * t)
                    else:
                        res = a * (dv * vbuf[i, sl]) + b * (dv * dv * t)
                    abuf[i, sl] = res
                return 0
            lax.fori_loop(0, RQ, row, 0)
            if final:
                pltpu.sync_copy(abuf, out.at[pl.ds(uoff + r, RQ)])
            else:
                pltpu.sync_copy(abuf, u_flat.at[pl.ds(uoff + r, RQ)])

    def hop(h, _):
        scatter_phase()
        plsc.subcore_barrier()
        return 0
    lax.fori_loop(0, HOPS - 1, hop, 0)

    scatter_phase()
    plsc.subcore_barrier()
    update_phase(final=True)


@jax.jit
def kernel(x, y, edge_index):
    v0 = jnp.concatenate([x, y], axis=1)
    zp = jnp.zeros((NPAD - N, F), jnp.float32)
    v0f = jnp.concatenate([v0[:, :F], zp, v0[:, F:], zp], axis=0)
    src = edge_index[0]
    dst = edge_index[1]
    pad = EPAD - E
    src = jnp.concatenate([src, jnp.zeros((pad,), jnp.int32)])
    dst = jnp.concatenate([dst, jnp.zeros((pad,), jnp.int32)])
    srcs = src.reshape(NS, NCHUNK, CHUNK)
    dsts = dst.reshape(NS, NCHUNK, CHUNK)

    mesh = plsc.VectorSubcoreMesh(core_axis_name="c", subcore_axis_name="s",
                                  num_cores=NC, num_subcores=NS)
    run = pl.kernel(
        _body,
        out_type=jax.ShapeDtypeStruct((NC * NPAD, F), jnp.float32),
        mesh=mesh,
        compiler_params=pltpu.CompilerParams(use_tc_tiling_on_sc=False),
        scratch_types=[
            pltpu.HBM((NC * NPAD, F), jnp.float32),
            pltpu.HBM((NC, NS, NCHUNK, CHUNK), jnp.int32),
            pltpu.HBM((NC, NS, NCHUNK, CHUNK), jnp.int32),
            pltpu.VMEM_SHARED((NPAD, F), jnp.float32),
            pltpu.VMEM_SHARED((NPAD, L), jnp.float32),
            pltpu.VMEM((CHUNK,), jnp.int32),
            pltpu.VMEM((CHUNK,), jnp.int32),
            pltpu.VMEM((CHUNK, F), jnp.float32),
            pltpu.VMEM((CHUNK, L), jnp.float32),
            pltpu.VMEM((RQ, F), jnp.float32),
            pltpu.VMEM((RQ, F), jnp.float32),
            pltpu.VMEM((RQ, F), jnp.float32),
            pltpu.VMEM((RQ, F), jnp.float32),
            pltpu.SemaphoreType.DMA,
        ],
    )
    vout = run(v0f, srcs, dsts)
    A, B = vout[:N], vout[NPAD:NPAD + N]
    x_out = jnp.concatenate([A, B[:, : D - F]], axis=1)
    y_out = B[:, D - F:]
    return (x_out, y_out)


# speedup vs baseline: 1.5507x; 1.5507x over previous
import jax
import jax.numpy as jnp
from jax import lax
from jax.experimental import pallas as pl
from jax.experimental.pallas import tpu as pltpu
from jax.experimental.pallas import tpu_sc as plsc

N = 10000
D = 128
E = 320000
HOPS = 8
ALPHA = 0.1

NC, NS, L = 2, 16, 16
F = 96
CHUNK = 128
EPT = 20096
NCHUNK = EPT // CHUNK
EPAD = EPT * NS
NPT = 640
RQ = 64
NQ = NPT // RQ
NPAD = NS * NPT
ZROW = N


def _rsqrt16(x):
    i = lax.bitcast_convert_type(x, jnp.int32)
    i = jnp.int32(0x5F3759DF) - lax.shift_right_logical(i, 1)
    y = lax.bitcast_convert_type(i, jnp.float32)
    for _ in range(3):
        y = y * (1.5 - 0.5 * x * y * y)
    return y


def _body(v0f, srcs, dsts, out, u_flat, srcp, dstp, agg, dinvrep,
          sbuf, dbuf, gbuf, dvbuf, abuf, ubuf, vbuf, zbuf, gsem, isem):
    c = lax.axis_index("c")
    s = lax.axis_index("s")
    uoff = c * NPAD
    rb = s * NPT

    zero16 = jnp.zeros((L,), jnp.float32)
    one16 = jnp.full((L,), 1.0, jnp.float32)

    def zero_row96(i, _):
        for kk in range(F // L):
            zbuf[i, pl.ds(kk * L, L)] = zero16
        return 0
    lax.fori_loop(0, RQ, zero_row96, 0)

    def zero_row16(i, _):
        dvbuf[i, :] = zero16
        return 0
    lax.fori_loop(0, CHUNK, zero_row16, 0)

    for q in range(NPT // CHUNK):
        pltpu.sync_copy(dvbuf, dinvrep.at[pl.ds(rb + q * CHUNK, CHUNK)])
    for q in range(NQ):
        pltpu.sync_copy(zbuf, agg.at[pl.ds(rb + q * RQ, RQ)])
    plsc.subcore_barrier()

    def ones_row(i, _):
        dvbuf[i, :] = one16
        return 0
    lax.fori_loop(0, CHUNK, ones_row, 0)

    coff16 = jnp.broadcast_to(uoff, (L,)).astype(jnp.int32)

    def deg_chunk(j, _):
        pltpu.sync_copy(srcs.at[s, j], sbuf.at[0])
        pltpu.sync_copy(dsts.at[s, j], dbuf.at[0])
        for kk in range(CHUNK // L):
            sl = pl.ds(kk * L, L)
            s16 = sbuf[0, sl]
            d16 = dbuf[0, sl]
            valid = s16 != d16
            sbuf[0, sl] = jnp.where(valid, s16, ZROW) + coff16
            dbuf[0, sl] = jnp.where(valid, d16, ZROW)
        pltpu.sync_copy(dvbuf, dinvrep.at[dbuf.at[0]], add=True)
        pltpu.sync_copy(sbuf.at[0], srcp.at[c, s, j])
        pltpu.sync_copy(dbuf.at[0], dstp.at[c, s, j])
        return 0
    lax.fori_loop(0, NCHUNK, deg_chunk, 0)
    plsc.subcore_barrier()

    def dinv_chunk(q, _):
        r = rb + q * RQ
        pltpu.sync_copy(dinvrep.at[pl.ds(r, RQ)], dvbuf.at[pl.ds(0, RQ)])

        def drow(i, _):
            dvbuf[i, :] = _rsqrt16(dvbuf[i, :] + 1.0)
            return 0
        lax.fori_loop(0, RQ, drow, 0)
        pltpu.sync_copy(dvbuf.at[pl.ds(0, RQ)], dinvrep.at[pl.ds(r, RQ)])
        return 0
    lax.fori_loop(0, NQ, dinv_chunk, 0)
    plsc.subcore_barrier()

    for q in range(NQ):
        r = rb + q * RQ
        pltpu.sync_copy(v0f.at[pl.ds(uoff + r, RQ)], vbuf)
        pltpu.sync_copy(dinvrep.at[pl.ds(r, RQ)], dvbuf.at[pl.ds(0, RQ)])

        def scale_row(i, _):
            dv = dvbuf[i, :]
            for kk in range(F // L):
                sl = pl.ds(kk * L, L)
                vbuf[i, sl] = vbuf[i, sl] * dv
            return 0
        lax.fori_loop(0, RQ, scale_row, 0)
        pltpu.sync_copy(vbuf, u_flat.at[pl.ds(uoff + r, RQ)])
    plsc.subcore_barrier()

    def scatter_phase():
        pltpu.sync_copy(srcp.at[c, s, 0], sbuf.at[0])
        pltpu.sync_copy(dstp.at[c, s, 0], dbuf.at[0])
        pltpu.async_copy(srcp.at[c, s, 1], sbuf.at[1], isem.at[1])
        pltpu.async_copy(dstp.at[c, s, 1], dbuf.at[1], isem.at[1])
        pltpu.async_copy(u_flat.at[sbuf.at[0]], gbuf.at[0], gsem.at[0])

        def chunk(k, _):
            p = lax.rem(k, 2)
            pn = 1 - p

            @pl.when(k + 1 < NCHUNK)
            def _():
                pltpu.make_async_copy(srcp.at[c, s, k + 1], sbuf.at[pn],
                                      isem.at[pn]).wait()
                pltpu.make_async_copy(dstp.at[c, s, k + 1], dbuf.at[pn],
                                      isem.at[pn]).wait()
                pltpu.async_copy(u_flat.at[sbuf.at[pn]], gbuf.at[pn],
                                 gsem.at[pn])

            pltpu.make_async_copy(u_flat.at[sbuf.at[p]], gbuf.at[p],
                                  gsem.at[p]).wait()
            pltpu.sync_copy(gbuf.at[p], agg.at[dbuf.at[p]], add=True)

            @pl.when(k + 2 < NCHUNK)
            def _():
                pltpu.async_copy(srcp.at[c, s, k + 2], sbuf.at[p], isem.at[p])
                pltpu.async_copy(dstp.at[c, s, k + 2], dbuf.at[p], isem.at[p])
            return 0
        lax.fori_loop(0, NCHUNK, chunk, 0)

    def update_phase(final):
        a, b = jnp.float32(ALPHA), jnp.float32(1.0 - ALPHA)
        for q in range(NQ):
            r = rb + q * RQ
            pltpu.sync_copy(agg.at[pl.ds(r, RQ)], abuf)
            pltpu.sync_copy(zbuf, agg.at[pl.ds(r, RQ)])
            pltpu.sync_copy(u_flat.at[pl.ds(uoff + r, RQ)], ubuf)
            pltpu.sync_copy(v0f.at[pl.ds(uoff + r, RQ)], vbuf)
            pltpu.sync_copy(dinvrep.at[pl.ds(r, RQ)], dvbuf.at[pl.ds(0, RQ)])

            def row(i, _):
                dv = dvbuf[i, :]
                for kk in range(F // L):
                    sl = pl.ds(kk * L, L)
                    t = abuf[i, sl] + ubuf[i, sl]
                    if final:
                        res = a * vbuf[i, sl] + b * (dv * t)
                    else:
                        res = a * (dv * vbuf[i, sl]) + b * (dv * dv * t)
                    abuf[i, sl] = res
                return 0
            lax.fori_loop(0, RQ, row, 0)
            if final:
                pltpu.sync_copy(abuf, out.at[pl.ds(uoff + r, RQ)])
            else:
                pltpu.sync_copy(abuf, u_flat.at[pl.ds(uoff + r, RQ)])

    def hop(h, _):
        scatter_phase()
        plsc.subcore_barrier()
        update_phase(final=False)
        plsc.subcore_barrier()
        return 0
    lax.fori_loop(0, HOPS - 1, hop, 0)

    scatter_phase()
    plsc.subcore_barrier()
    update_phase(final=True)


@jax.jit
def kernel(x, y, edge_index):
    v0 = jnp.concatenate([x, y], axis=1)
    zp = jnp.zeros((NPAD - N, F), jnp.float32)
    v0f = jnp.concatenate([v0[:, :F], zp, v0[:, F:], zp], axis=0)
    src = edge_index[0]
    dst = edge_index[1]
    pad = EPAD - E
    src = jnp.concatenate([src, jnp.zeros((pad,), jnp.int32)])
    dst = jnp.concatenate([dst, jnp.zeros((pad,), jnp.int32)])
    srcs = src.reshape(NS, NCHUNK, CHUNK)
    dsts = dst.reshape(NS, NCHUNK, CHUNK)

    mesh = plsc.VectorSubcoreMesh(core_axis_name="c", subcore_axis_name="s",
                                  num_cores=NC, num_subcores=NS)
    run = pl.kernel(
        _body,
        out_type=jax.ShapeDtypeStruct((NC * NPAD, F), jnp.float32),
        mesh=mesh,
        compiler_params=pltpu.CompilerParams(use_tc_tiling_on_sc=False),
        scratch_types=[
            pltpu.HBM((NC * NPAD, F), jnp.float32),
            pltpu.HBM((NC, NS, NCHUNK, CHUNK), jnp.int32),
            pltpu.HBM((NC, NS, NCHUNK, CHUNK), jnp.int32),
            pltpu.VMEM_SHARED((NPAD, F), jnp.float32),
            pltpu.VMEM_SHARED((NPAD, L), jnp.float32),
            pltpu.VMEM((2, CHUNK), jnp.int32),
            pltpu.VMEM((2, CHUNK), jnp.int32),
            pltpu.VMEM((2, CHUNK, F), jnp.float32),
            pltpu.VMEM((CHUNK, L), jnp.float32),
            pltpu.VMEM((RQ, F), jnp.float32),
            pltpu.VMEM((RQ, F), jnp.float32),
            pltpu.VMEM((RQ, F), jnp.float32),
            pltpu.VMEM((RQ, F), jnp.float32),
            pltpu.SemaphoreType.DMA((2,)),
            pltpu.SemaphoreType.DMA((2,)),
        ],
    )
    vout = run(v0f, srcs, dsts)
    A, B = vout[:N], vout[NPAD:NPAD + N]
    x_out = jnp.concatenate([A, B[:, : D - F]], axis=1)
    y_out = B[:, D - F:]
    return (x_out, y_out)


# speedup vs baseline: 1.6302x; 1.0513x over previous
import jax
import jax.numpy as jnp
from jax import lax
from jax.experimental import pallas as pl
from jax.experimental.pallas import tpu as pltpu
from jax.experimental.pallas import tpu_sc as plsc

N = 10000
D = 128
E = 320000
HOPS = 8
ALPHA = 0.1

NC, NS, L = 2, 16, 16
F = 96
CHUNK = 128
EPT = 20096
NCHUNK = EPT // CHUNK
EPAD = EPT * NS
NPT = 640
RQ = 64
NQ = NPT // RQ
NPAD = NS * NPT
ZROW = N


def _rsqrt16(x):
    i = lax.bitcast_convert_type(x, jnp.int32)
    i = jnp.int32(0x5F3759DF) - lax.shift_right_logical(i, 1)
    y = lax.bitcast_convert_type(i, jnp.float32)
    for _ in range(3):
        y = y * (1.5 - 0.5 * x * y * y)
    return y


def _body(v0f, srcs, dsts, out, u_flat, srcp, dstp, agg, dinvrep,
          sbuf, dbuf, gbuf, dvbuf, abuf, ubuf, vbuf, zbuf, gsem, isem):
    c = lax.axis_index("c")
    s = lax.axis_index("s")
    uoff = c * NPAD
    rb = s * NPT

    zero16 = jnp.zeros((L,), jnp.float32)
    one16 = jnp.full((L,), 1.0, jnp.float32)

    def zero_row96(i, _):
        for kk in range(F // L):
            zbuf[i, pl.ds(kk * L, L)] = zero16
        return 0
    lax.fori_loop(0, RQ, zero_row96, 0)

    def zero_row16(i, _):
        dvbuf[i, :] = zero16
        return 0
    lax.fori_loop(0, CHUNK, zero_row16, 0)

    for q in range(NPT // CHUNK):
        pltpu.sync_copy(dvbuf, dinvrep.at[pl.ds(rb + q * CHUNK, CHUNK)])
    for q in range(NQ):
        pltpu.sync_copy(zbuf, agg.at[pl.ds(rb + q * RQ, RQ)])
    plsc.subcore_barrier()

    def ones_row(i, _):
        dvbuf[i, :] = one16
        return 0
    lax.fori_loop(0, CHUNK, ones_row, 0)

    coff16 = jnp.broadcast_to(uoff, (L,)).astype(jnp.int32)

    pltpu.sync_copy(srcs.at[s, 0], sbuf.at[0])
    pltpu.sync_copy(dsts.at[s, 0], dbuf.at[0])
    pltpu.async_copy(srcs.at[s, 1], sbuf.at[1], isem.at[1])
    pltpu.async_copy(dsts.at[s, 1], dbuf.at[1], isem.at[1])

    def deg_chunk(j, _):
        p = lax.rem(j, 2)

        @pl.when(j >= 1)
        def _():
            pltpu.make_async_copy(srcs.at[s, j], sbuf.at[p], isem.at[p]).wait()
            pltpu.make_async_copy(dsts.at[s, j], dbuf.at[p], isem.at[p]).wait()

        for kk in range(CHUNK // L):
            sl = pl.ds(kk * L, L)
            s16 = sbuf[p, sl]
            d16 = dbuf[p, sl]
            valid = s16 != d16
            sbuf[p, sl] = jnp.where(valid, s16, ZROW) + coff16
            dbuf[p, sl] = jnp.where(valid, d16, ZROW)
        pltpu.sync_copy(dvbuf, dinvrep.at[dbuf.at[p]], add=True)
        pltpu.sync_copy(sbuf.at[p], srcp.at[c, s, j])
        pltpu.sync_copy(dbuf.at[p], dstp.at[c, s, j])

        @pl.when(j + 2 < NCHUNK)
        def _():
            pltpu.async_copy(srcs.at[s, j + 2], sbuf.at[p], isem.at[p])
            pltpu.async_copy(dsts.at[s, j + 2], dbuf.at[p], isem.at[p])
        return 0
    lax.fori_loop(0, NCHUNK, deg_chunk, 0)
    plsc.subcore_barrier()

    def dinv_chunk(q, _):
        r = rb + q * RQ
        pltpu.sync_copy(dinvrep.at[pl.ds(r, RQ)], dvbuf.at[pl.ds(0, RQ)])

        def drow(i, _):
            dvbuf[i, :] = _rsqrt16(dvbuf[i, :] + 1.0)
            return 0
        lax.fori_loop(0, RQ, drow, 0)
        pltpu.sync_copy(dvbuf.at[pl.ds(0, RQ)], dinvrep.at[pl.ds(r, RQ)])
        return 0
    lax.fori_loop(0, NQ, dinv_chunk, 0)
    plsc.subcore_barrier()

    for q in range(NQ):
        r = rb + q * RQ
        pltpu.sync_copy(v0f.at[pl.ds(uoff + r, RQ)], vbuf)
        pltpu.sync_copy(dinvrep.at[pl.ds(r, RQ)], dvbuf.at[pl.ds(0, RQ)])

        def scale_row(i, _):
            dv = dvbuf[i, :]
            for kk in range(F // L):
                sl = pl.ds(kk * L, L)
                vbuf[i, sl] = vbuf[i, sl] * dv
            return 0
        lax.fori_loop(0, RQ, scale_row, 0)
        pltpu.sync_copy(vbuf, u_flat.at[pl.ds(uoff + r, RQ)])
    plsc.subcore_barrier()

    def scatter_phase():
        pltpu.sync_copy(srcp.at[c, s, 0], sbuf.at[0])
        pltpu.sync_copy(dstp.at[c, s, 0], dbuf.at[0])
        pltpu.async_copy(srcp.at[c, s, 1], sbuf.at[1], isem.at[1])
        pltpu.async_copy(dstp.at[c, s, 1], dbuf.at[1], isem.at[1])
        pltpu.async_copy(u_flat.at[sbuf.at[0]], gbuf.at[0], gsem.at[0])

        def chunk(k, _):
            p = lax.rem(k, 2)
            pn = 1 - p

            @pl.when(k + 1 < NCHUNK)
            def _():
                pltpu.make_async_copy(srcp.at[c, s, k + 1], sbuf.at[pn],
                                      isem.at[pn]).wait()
                pltpu.make_async_copy(dstp.at[c, s, k + 1], dbuf.at[pn],
                                      isem.at[pn]).wait()
                pltpu.async_copy(u_flat.at[sbuf.at[pn]], gbuf.at[pn],
                                 gsem.at[pn])

            pltpu.make_async_copy(u_flat.at[sbuf.at[p]], gbuf.at[p],
                                  gsem.at[p]).wait()
            pltpu.sync_copy(gbuf.at[p], agg.at[dbuf.at[p]], add=True)

            @pl.when(k + 2 < NCHUNK)
            def _():
                pltpu.async_copy(srcp.at[c, s, k + 2], sbuf.at[p], isem.at[p])
                pltpu.async_copy(dstp.at[c, s, k + 2], dbuf.at[p], isem.at[p])
            return 0
        lax.fori_loop(0, NCHUNK, chunk, 0)

    def update_phase(final):
        a, b = jnp.float32(ALPHA), jnp.float32(1.0 - ALPHA)
        for q in range(NQ):
            r = rb + q * RQ
            pltpu.sync_copy(agg.at[pl.ds(r, RQ)], abuf)
            pltpu.sync_copy(zbuf, agg.at[pl.ds(r, RQ)])
            pltpu.sync_copy(u_flat.at[pl.ds(uoff + r, RQ)], ubuf)
            pltpu.sync_copy(v0f.at[pl.ds(uoff + r, RQ)], vbuf)
            pltpu.sync_copy(dinvrep.at[pl.ds(r, RQ)], dvbuf.at[pl.ds(0, RQ)])

            def row(i, _):
                dv = dvbuf[i, :]
                for kk in range(F // L):
                    sl = pl.ds(kk * L, L)
                    t = abuf[i, sl] + ubuf[i, sl]
                    if final:
                        res = a * vbuf[i, sl] + b * (dv * t)
                    else:
                        res = a * (dv * vbuf[i, sl]) + b * (dv * dv * t)
                    abuf[i, sl] = res
                return 0
            lax.fori_loop(0, RQ, row, 0)
            if final:
                pltpu.sync_copy(abuf, out.at[pl.ds(uoff + r, RQ)])
            else:
                pltpu.sync_copy(abuf, u_flat.at[pl.ds(uoff + r, RQ)])

    def hop(h, _):
        scatter_phase()
        plsc.subcore_barrier()
        update_phase(final=False)
        plsc.subcore_barrier()
        return 0
    lax.fori_loop(0, HOPS - 1, hop, 0)

    scatter_phase()
    plsc.subcore_barrier()
    update_phase(final=True)


@jax.jit
def kernel(x, y, edge_index):
    v0 = jnp.concatenate([x, y], axis=1)
    zp = jnp.zeros((NPAD - N, F), jnp.float32)
    v0f = jnp.concatenate([v0[:, :F], zp, v0[:, F:], zp], axis=0)
    src = edge_index[0]
    dst = edge_index[1]
    pad = EPAD - E
    src = jnp.concatenate([src, jnp.zeros((pad,), jnp.int32)])
    dst = jnp.concatenate([dst, jnp.zeros((pad,), jnp.int32)])
    srcs = src.reshape(NS, NCHUNK, CHUNK)
    dsts = dst.reshape(NS, NCHUNK, CHUNK)

    mesh = plsc.VectorSubcoreMesh(core_axis_name="c", subcore_axis_name="s",
                                  num_cores=NC, num_subcores=NS)
    run = pl.kernel(
        _body,
        out_type=jax.ShapeDtypeStruct((NC * NPAD, F), jnp.float32),
        mesh=mesh,
        compiler_params=pltpu.CompilerParams(use_tc_tiling_on_sc=False),
        scratch_types=[
            pltpu.HBM((NC * NPAD, F), jnp.float32),
            pltpu.HBM((NC, NS, NCHUNK, CHUNK), jnp.int32),
            pltpu.HBM((NC, NS, NCHUNK, CHUNK), jnp.int32),
            pltpu.VMEM_SHARED((NPAD, F), jnp.float32),
            pltpu.VMEM_SHARED((NPAD, L), jnp.float32),
            pltpu.VMEM((2, CHUNK), jnp.int32),
            pltpu.VMEM((2, CHUNK), jnp.int32),
            pltpu.VMEM((2, CHUNK, F), jnp.float32),
            pltpu.VMEM((CHUNK, L), jnp.float32),
            pltpu.VMEM((RQ, F), jnp.float32),
            pltpu.VMEM((RQ, F), jnp.float32),
            pltpu.VMEM((RQ, F), jnp.float32),
            pltpu.VMEM((RQ, F), jnp.float32),
            pltpu.SemaphoreType.DMA((2,)),
            pltpu.SemaphoreType.DMA((2,)),
        ],
    )
    vout = run(v0f, srcs, dsts)
    A, B = vout[:N], vout[NPAD:NPAD + N]
    x_out = jnp.concatenate([A, B[:, : D - F]], axis=1)
    y_out = B[:, D - F:]
    return (x_out, y_out)


# speedup vs baseline: 1.7853x; 1.0951x over previous
import jax
import jax.numpy as jnp
from jax import lax
from jax.experimental import pallas as pl
from jax.experimental.pallas import tpu as pltpu
from jax.experimental.pallas import tpu_sc as plsc

N = 10000
D = 128
E = 320000
HOPS = 8
ALPHA = 0.1

NC, NS, L = 2, 16, 16
F = 96
CHUNK = 128
EPT = 20096
NCHUNK = EPT // CHUNK
EPAD = EPT * NS
NPT = 640
RQ = 64
NQ = NPT // RQ
NPAD = NS * NPT
ZROW = N


def _rsqrt16(x):
    i = lax.bitcast_convert_type(x, jnp.int32)
    i = jnp.int32(0x5F3759DF) - lax.shift_right_logical(i, 1)
    y = lax.bitcast_convert_type(i, jnp.float32)
    for _ in range(3):
        y = y * (1.5 - 0.5 * x * y * y)
    return y


def _body(v0f, srcs, dsts, out, u_flat, srcp, dstp, agg, dinvrep,
          sbuf, dbuf, gbuf, dvbuf, abuf, ubuf, vbuf, zbuf, gsem, isem, ssem):
    c = lax.axis_index("c")
    s = lax.axis_index("s")
    uoff = c * NPAD
    rb = s * NPT

    zero16 = jnp.zeros((L,), jnp.float32)
    one16 = jnp.full((L,), 1.0, jnp.float32)

    def zero_row96(i, _):
        for kk in range(F // L):
            zbuf[i, pl.ds(kk * L, L)] = zero16
        return 0
    lax.fori_loop(0, RQ, zero_row96, 0)

    def zero_row16(i, _):
        dvbuf[i, :] = zero16
        return 0
    lax.fori_loop(0, CHUNK, zero_row16, 0)

    for q in range(NPT // CHUNK):
        pltpu.sync_copy(dvbuf, dinvrep.at[pl.ds(rb + q * CHUNK, CHUNK)])
    for q in range(NQ):
        pltpu.sync_copy(zbuf, agg.at[pl.ds(rb + q * RQ, RQ)])
    plsc.subcore_barrier()

    def ones_row(i, _):
        dvbuf[i, :] = one16
        return 0
    lax.fori_loop(0, CHUNK, ones_row, 0)

    coff16 = jnp.broadcast_to(uoff, (L,)).astype(jnp.int32)

    pltpu.sync_copy(srcs.at[s, 0], sbuf.at[0])
    pltpu.sync_copy(dsts.at[s, 0], dbuf.at[0])
    pltpu.async_copy(srcs.at[s, 1], sbuf.at[1], isem.at[1])
    pltpu.async_copy(dsts.at[s, 1], dbuf.at[1], isem.at[1])

    def deg_chunk(j, _):
        p = lax.rem(j, 2)

        @pl.when(j >= 1)
        def _():
            pltpu.make_async_copy(srcs.at[s, j], sbuf.at[p], isem.at[p]).wait()
            pltpu.make_async_copy(dsts.at[s, j], dbuf.at[p], isem.at[p]).wait()

        for kk in range(CHUNK // L):
            sl = pl.ds(kk * L, L)
            s16 = sbuf[p, sl]
            d16 = dbuf[p, sl]
            valid = s16 != d16
            sbuf[p, sl] = jnp.where(valid, s16, ZROW) + coff16
            dbuf[p, sl] = jnp.where(valid, d16, ZROW)
        pltpu.sync_copy(dvbuf, dinvrep.at[dbuf.at[p]], add=True)
        pltpu.sync_copy(sbuf.at[p], srcp.at[c, s, j])
        pltpu.sync_copy(dbuf.at[p], dstp.at[c, s, j])

        @pl.when(j + 2 < NCHUNK)
        def _():
            pltpu.async_copy(srcs.at[s, j + 2], sbuf.at[p], isem.at[p])
            pltpu.async_copy(dsts.at[s, j + 2], dbuf.at[p], isem.at[p])
        return 0
    lax.fori_loop(0, NCHUNK, deg_chunk, 0)
    plsc.subcore_barrier()

    def dinv_chunk(q, _):
        r = rb + q * RQ
        pltpu.sync_copy(dinvrep.at[pl.ds(r, RQ)], dvbuf.at[pl.ds(0, RQ)])

        def drow(i, _):
            dvbuf[i, :] = _rsqrt16(dvbuf[i, :] + 1.0)
            return 0
        lax.fori_loop(0, RQ, drow, 0)
        pltpu.sync_copy(dvbuf.at[pl.ds(0, RQ)], dinvrep.at[pl.ds(r, RQ)])
        return 0
    lax.fori_loop(0, NQ, dinv_chunk, 0)
    plsc.subcore_barrier()

    for q in range(NQ):
        r = rb + q * RQ
        pltpu.sync_copy(v0f.at[pl.ds(uoff + r, RQ)], vbuf)
        pltpu.sync_copy(dinvrep.at[pl.ds(r, RQ)], dvbuf.at[pl.ds(0, RQ)])

        def scale_row(i, _):
            dv = dvbuf[i, :]
            for kk in range(F // L):
                sl = pl.ds(kk * L, L)
                vbuf[i, sl] = vbuf[i, sl] * dv
            return 0
        lax.fori_loop(0, RQ, scale_row, 0)
        pltpu.sync_copy(vbuf, u_flat.at[pl.ds(uoff + r, RQ)])
    plsc.subcore_barrier()

    def scatter_phase():
        pltpu.sync_copy(srcp.at[c, s, 0], sbuf.at[0])
        pltpu.sync_copy(dstp.at[c, s, 0], dbuf.at[0])
        pltpu.async_copy(srcp.at[c, s, 1], sbuf.at[1], isem.at[1])
        pltpu.async_copy(dstp.at[c, s, 1], dbuf.at[1], isem.at[1])
        pltpu.async_copy(u_flat.at[sbuf.at[0]], gbuf.at[0], gsem.at[0])

        def chunk(k, _):
            p = lax.rem(k, 2)
            pn = 1 - p
            m = lax.rem(k, 3)

            @pl.when(k >= 1)
            def _():
                pltpu.make_async_copy(gbuf.at[pn],
                                      agg.at[dbuf.at[lax.rem(k - 1, 3)]],
                                      ssem).wait()

            @pl.when(k + 1 < NCHUNK)
            def _():
                m1 = lax.rem(k + 1, 3)
                pltpu.make_async_copy(srcp.at[c, s, k + 1], sbuf.at[m1],
                                      isem.at[m1]).wait()
                pltpu.make_async_copy(dstp.at[c, s, k + 1], dbuf.at[m1],
                                      isem.at[m1]).wait()
                pltpu.async_copy(u_flat.at[sbuf.at[m1]], gbuf.at[pn],
                                 gsem.at[pn])

            pltpu.make_async_copy(u_flat.at[sbuf.at[m]], gbuf.at[p],
                                  gsem.at[p]).wait()
            pltpu.async_copy(gbuf.at[p], agg.at[dbuf.at[m]], ssem, add=True)

            @pl.when(k + 2 < NCHUNK)
            def _():
                m2 = lax.rem(k + 2, 3)
                pltpu.async_copy(srcp.at[c, s, k + 2], sbuf.at[m2],
                                 isem.at[m2])
                pltpu.async_copy(dstp.at[c, s, k + 2], dbuf.at[m2],
                                 isem.at[m2])
            return 0
        lax.fori_loop(0, NCHUNK, chunk, 0)
        pltpu.make_async_copy(gbuf.at[(NCHUNK - 1) % 2],
                              agg.at[dbuf.at[(NCHUNK - 1) % 3]], ssem).wait()

    def update_phase(final):
        a, b = jnp.float32(ALPHA), jnp.float32(1.0 - ALPHA)
        for q in range(NQ):
            r = rb + q * RQ
            pltpu.sync_copy(agg.at[pl.ds(r, RQ)], abuf)
            pltpu.sync_copy(zbuf, agg.at[pl.ds(r, RQ)])
            pltpu.sync_copy(u_flat.at[pl.ds(uoff + r, RQ)], ubuf)
            pltpu.sync_copy(v0f.at[pl.ds(uoff + r, RQ)], vbuf)
            pltpu.sync_copy(dinvrep.at[pl.ds(r, RQ)], dvbuf.at[pl.ds(0, RQ)])

            def row(i, _):
                dv = dvbuf[i, :]
                for kk in range(F // L):
                    sl = pl.ds(kk * L, L)
                    t = abuf[i, sl] + ubuf[i, sl]
                    if final:
                        res = a * vbuf[i, sl] + b * (dv * t)
                    else:
                        res = a * (dv * vbuf[i, sl]) + b * (dv * dv * t)
                    abuf[i, sl] = res
                return 0
            lax.fori_loop(0, RQ, row, 0)
            if final:
                pltpu.sync_copy(abuf, out.at[pl.ds(uoff + r, RQ)])
            else:
                pltpu.sync_copy(abuf, u_flat.at[pl.ds(uoff + r, RQ)])

    def hop(h, _):
        scatter_phase()
        plsc.subcore_barrier()
        update_phase(final=False)
        plsc.subcore_barrier()
        return 0
    lax.fori_loop(0, HOPS - 1, hop, 0)

    scatter_phase()
    plsc.subcore_barrier()
    update_phase(final=True)


@jax.jit
def kernel(x, y, edge_index):
    v0 = jnp.concatenate([x, y], axis=1)
    zp = jnp.zeros((NPAD - N, F), jnp.float32)
    v0f = jnp.concatenate([v0[:, :F], zp, v0[:, F:], zp], axis=0)
    src = edge_index[0]
    dst = edge_index[1]
    pad = EPAD - E
    src = jnp.concatenate([src, jnp.zeros((pad,), jnp.int32)])
    dst = jnp.concatenate([dst, jnp.zeros((pad,), jnp.int32)])
    srcs = src.reshape(NS, NCHUNK, CHUNK)
    dsts = dst.reshape(NS, NCHUNK, CHUNK)

    mesh = plsc.VectorSubcoreMesh(core_axis_name="c", subcore_axis_name="s",
                                  num_cores=NC, num_subcores=NS)
    run = pl.kernel(
        _body,
        out_type=jax.ShapeDtypeStruct((NC * NPAD, F), jnp.float32),
        mesh=mesh,
        compiler_params=pltpu.CompilerParams(use_tc_tiling_on_sc=False),
        scratch_types=[
            pltpu.HBM((NC * NPAD, F), jnp.float32),
            pltpu.HBM((NC, NS, NCHUNK, CHUNK), jnp.int32),
            pltpu.HBM((NC, NS, NCHUNK, CHUNK), jnp.int32),
            pltpu.VMEM_SHARED((NPAD, F), jnp.float32),
            pltpu.VMEM_SHARED((NPAD, L), jnp.float32),
            pltpu.VMEM((3, CHUNK), jnp.int32),
            pltpu.VMEM((3, CHUNK), jnp.int32),
            pltpu.VMEM((2, CHUNK, F), jnp.float32),
            pltpu.VMEM((CHUNK, L), jnp.float32),
            pltpu.VMEM((RQ, F), jnp.float32),
            pltpu.VMEM((RQ, F), jnp.float32),
            pltpu.VMEM((RQ, F), jnp.float32),
            pltpu.VMEM((RQ, F), jnp.float32),
            pltpu.SemaphoreType.DMA((2,)),
            pltpu.SemaphoreType.DMA((3,)),
            pltpu.SemaphoreType.DMA,
        ],
    )
    vout = run(v0f, srcs, dsts)
    A, B = vout[:N], vout[NPAD:NPAD + N]
    x_out = jnp.concatenate([A, B[:, : D - F]], axis=1)
    y_out = B[:, D - F:]
    return (x_out, y_out)
